# Initial kernel scaffold; baseline (speedup 1.0000x reference)
import jax
import jax.numpy as jnp
from jax.experimental import pallas as pl


def kernel(x, mask, edge_index, edge_type, reliable_masking, emb_W, emb_b, mask_tok, enc_gcn_W, enc_gcn_b, enc_bn1_g, enc_bn1_b, enc_bn2_g, enc_bn2_b, enc_l1_W, enc_l1_b, enc_l2_W, enc_l2_b, dec_gcn_W, dec_gcn_b, dec_bn1_g, dec_bn1_b, dec_bn2_g, dec_bn2_b, dec_l1_W, dec_l1_b, dec_l2_W, dec_l2_b):
    raise NotImplementedError("write your pallas kernel here")


# speedup vs baseline: 1.2384x; 1.2384x over previous
import functools
import jax
import jax.numpy as jnp
from jax.experimental import pallas as pl

_N = 10000
_H = 256
_T = 4
_EPS = 1e-5
_NUM_ITER = 3


def _emb_body(x_ref, w_ref, b_ref, o_ref):
    o_ref[...] = x_ref[...] @ w_ref[...] + b_ref[...]


def _emb(x2d, W, b):
    return pl.pallas_call(
        _emb_body,
        out_shape=jax.ShapeDtypeStruct((_N, _H), jnp.float32),
    )(x2d, W, b[None, :])


def _bn(x, g, b):
    m = jnp.mean(x, axis=0)
    v = jnp.var(x, axis=0)
    return (x - m) / jnp.sqrt(v + _EPS) * g + b


def _stack(x, src, dst, etype, dis, gcn_W, gcn_b, bn1_g, bn1_b, bn2_g, bn2_b,
           l1_W, l1_b, l2_W, l2_b):
    L = gcn_W.shape[0]
    for i in range(L):
        x2 = jnp.zeros_like(x)
        for t in range(_T):
            G = (x @ gcn_W[i, t]) * dis[t][:, None]
            valid = etype == t
            msg = jnp.where(valid[:, None], G[src], -jnp.inf)
            M = jnp.full((_N, _H), -jnp.inf, x.dtype).at[dst].max(msg)
            M = jnp.maximum(M, G)
            x2 = x2 + dis[t][:, None] * M + gcn_b[i, t]
        x = x + x2
        x = _bn(x, bn1_g[i], bn1_b[i])
        x = jax.nn.relu(x @ l1_W[i] + l1_b[i]) @ l2_W[i] + l2_b[i]
        x = _bn(x, bn2_g[i], bn2_b[i])
        x = jax.nn.relu(x)
    return x


def kernel(x, mask, edge_index, edge_type, reliable_masking, emb_W, emb_b, mask_tok,
           enc_gcn_W, enc_gcn_b, enc_bn1_g, enc_bn1_b, enc_bn2_g, enc_bn2_b,
           enc_l1_W, enc_l1_b, enc_l2_W, enc_l2_b,
           dec_gcn_W, dec_gcn_b, dec_bn1_g, dec_bn1_b, dec_bn2_g, dec_bn2_b,
           dec_l1_W, dec_l1_b, dec_l2_W, dec_l2_b):
    src = edge_index[0]
    dst = edge_index[1]
    et = edge_type
    deg = jnp.zeros((_T, _N), jnp.float32).at[et, dst].add(1.0) + 1.0
    dis = jax.lax.rsqrt(deg)
    h = _emb(x[:, 0, :], emb_W, emb_b)
    h = jnp.where(mask, mask_tok[None, :], h)
    h = _stack(h, src, dst, et, dis, enc_gcn_W, enc_gcn_b, enc_bn1_g, enc_bn1_b,
               enc_bn2_g, enc_bn2_b, enc_l1_W, enc_l1_b, enc_l2_W, enc_l2_b)
    noise = jax.random.normal(jax.random.key(42), (_N, _H), dtype=jnp.float32)
    nmask = jnp.zeros((_H,), jnp.float32).at[: _H // 2].set(1.0)
    h = h + nmask[None, :] * noise
    for _ in range(_NUM_ITER):
        out = _stack(h, src, dst, et, dis, dec_gcn_W, dec_gcn_b, dec_bn1_g, dec_bn1_b,
                     dec_bn2_g, dec_bn2_b, dec_l1_W, dec_l1_b, dec_l2_W, dec_l2_b)
        h = out + h
    return h


# speedup vs baseline: 3.5860x; 2.8956x over previous
import functools

import jax
import jax.numpy as jnp
from jax import lax
from jax.experimental import pallas as pl
from jax.experimental.pallas import tpu as pltpu
from jax.experimental.pallas import tpu_sc as plsc

N = 10000
NPAD = 10240
RANGE = 320
NW = 32
T = 4
F = 128
H = 256
DI = 4 * H
EPS = 1e-5
NUM_ITER = 3

EP = 163840
EW = EP // NW
ES = 164928
NBK = T * NW
CH = 32
BLK = 256
NBLK = NPAD // BLK


def _wid():
    return lax.axis_index("s") * 2 + lax.axis_index("c")


def _mesh():
    return plsc.VectorSubcoreMesh(core_axis_name="c", subcore_axis_name="s")


def _sortA_body(dst_hbm, et_hbm, tot_hbm, dstb, etb, cnt, totb):
    wid = _wid()
    iota = lax.iota(jnp.int32, 16)

    def zero(i, _):
        cnt[pl.ds(pl.multiple_of(i * 16, 8), 16)] = jnp.zeros((16,), jnp.int32)
        return 0

    lax.fori_loop(0, NBK, zero, 0, unroll=False)
    for chk in range(5):
        base = pl.multiple_of(wid * EW + chk * 1024, 8)
        pltpu.sync_copy(dst_hbm.at[pl.ds(base, 1024)], dstb)
        pltpu.sync_copy(et_hbm.at[pl.ds(base, 1024)], etb)

        def upd(v, _):
            o = pl.multiple_of(v * 16, 8)
            d = dstb[pl.ds(o, 16)]
            t = etb[pl.ds(o, 16)]
            b = t * NW + lax.div(d, RANGE)
            plsc.addupdate_scatter(cnt, [b * 16 + iota], jnp.ones((16,), jnp.int32))
            return 0

        lax.fori_loop(0, 64, upd, 0, unroll=False)
    for g in range(NBK // 16):
        bb = g * 16 + iota
        acc = jnp.zeros((16,), jnp.int32)
        for L in range(16):
            acc = acc + plsc.load_gather(cnt, [bb * 16 + L])
        totb[pl.ds(g * 16, 16)] = acc
    pltpu.sync_copy(totb, tot_hbm.at[pl.ds(pl.multiple_of(wid * NBK, 8), NBK)])


def _make_sortA():
    return pl.kernel(
        _sortA_body,
        out_type=jax.ShapeDtypeStruct((NW * NBK,), jnp.int32),
        compiler_params=pltpu.CompilerParams(needs_layout_passes=False),
        mesh=_mesh(),
        scratch_types=[
            pltpu.VMEM((1024,), jnp.int32),
            pltpu.VMEM((1024,), jnp.int32),
            pltpu.VMEM((NBK * 16,), jnp.int32),
            pltpu.VMEM((NBK,), jnp.int32),
        ],
    )


def _sortB_body(dst_hbm, et_hbm, src_hbm, tot_hbm,
                sg_hbm, sd_hbm, stw_hbm,
                srcb, dstb, etb, rowb, totall, gstart, gend, wbase,
                cnt, nextpos, posb, sgb, sdb, stwb, sem, sem2):
    wid = _wid()
    iota = lax.iota(jnp.int32, 16)
    zeros = jnp.zeros((16,), jnp.int32)
    for g in range(8):
        totall[pl.ds(g * 16, 16)] = zeros
        wbase[pl.ds(g * 16, 16)] = zeros
    for w in range(NW):
        pltpu.sync_copy(tot_hbm.at[pl.ds(w * NBK, NBK)], rowb)
        for g in range(8):
            r = rowb[pl.ds(g * 16, 16)]
            totall[pl.ds(g * 16, 16)] = totall[pl.ds(g * 16, 16)] + r
            wbase[pl.ds(g * 16, 16)] = wbase[pl.ds(g * 16, 16)] + jnp.where(w < wid, r, 0)
    carry = jnp.int32(0)
    for g in range(8):
        tv = totall[pl.ds(g * 16, 16)]
        av = (tv + 7) & ~7
        incl = plsc.cumsum(av)
        excl = incl - av + carry
        gstart[pl.ds(g * 16, 16)] = excl
        gend[pl.ds(g * 16, 16)] = excl + tv
        wbase[pl.ds(g * 16, 16)] = wbase[pl.ds(g * 16, 16)] + excl
        carry = carry + jnp.sum(av)
    idxh = lax.div(iota, 2) * NW + wid
    gA = plsc.load_gather(gstart, [idxh])
    gB = plsc.load_gather(gend, [idxh])
    stwb[...] = jnp.where((iota & 1) == 0, gA, gB)
    pltpu.sync_copy(stwb, stw_hbm.at[pl.ds(pl.multiple_of(wid * 16, 8), 16)])
    def zero(i, _):
        cnt[pl.ds(pl.multiple_of(i * 16, 8), 16)] = jnp.zeros((16,), jnp.int32)
        return 0

    lax.fori_loop(0, NBK, zero, 0, unroll=False)
    for chk in range(5):
        base = pl.multiple_of(wid * EW + chk * 1024, 8)
        pltpu.sync_copy(dst_hbm.at[pl.ds(base, 1024)], dstb)
        pltpu.sync_copy(et_hbm.at[pl.ds(base, 1024)], etb)

        def upd(v, _):
            o = pl.multiple_of(v * 16, 8)
            d = dstb[pl.ds(o, 16)]
            t = etb[pl.ds(o, 16)]
            b = t * NW + lax.div(d, RANGE)
            plsc.addupdate_scatter(cnt, [b * 16 + iota], jnp.ones((16,), jnp.int32))
            return 0

        lax.fori_loop(0, 64, upd, 0, unroll=False)
    for b in range(NBK):
        wv = wbase[pl.ds((b // 16) * 16, 16)]
        ws = wv[b % 16]
        row = cnt[pl.ds(b * 16, 16)]
        incl = plsc.cumsum(row)
        nextpos[pl.ds(b * 16, 16)] = incl - row + ws
    for chk in range(5):
        base = pl.multiple_of(wid * EW + chk * 1024, 8)
        pltpu.sync_copy(src_hbm.at[pl.ds(base, 1024)], srcb)
        pltpu.sync_copy(dst_hbm.at[pl.ds(base, 1024)], dstb)
        pltpu.sync_copy(et_hbm.at[pl.ds(base, 1024)], etb)

        def batch(bt, _):
            for k in range(8):
                o = pl.multiple_of((bt * 8 + k) * 16, 8)
                d = dstb[pl.ds(o, 16)]
                t = etb[pl.ds(o, 16)]
                s = srcb[pl.ds(o, 16)]
                b = t * NW + lax.div(d, RANGE)
                bi = b * 16 + iota
                pos = plsc.load_gather(nextpos, [bi])
                plsc.store_scatter(nextpos, [bi], pos + 1)
                posb[pl.ds(k * 16, 16)] = pos
                sgb[pl.ds(k * 16, 16)] = t * NPAD + s
                sdb[pl.ds(k * 16, 16)] = d
            c1 = pltpu.async_copy(sgb, sg_hbm.at[posb], sem)
            c2 = pltpu.async_copy(sdb, sd_hbm.at[posb], sem2)
            c1.wait()
            c2.wait()
            return 0

        lax.fori_loop(0, 8, batch, 0, unroll=False)


def _make_sortB():
    return pl.kernel(
        _sortB_body,
        out_type=(
            jax.ShapeDtypeStruct((ES,), jnp.int32),
            jax.ShapeDtypeStruct((ES,), jnp.int32),
            jax.ShapeDtypeStruct((NW * 16,), jnp.int32),
        ),
        compiler_params=pltpu.CompilerParams(needs_layout_passes=False),
        mesh=_mesh(),
        scratch_types=[
            pltpu.VMEM((1024,), jnp.int32),
            pltpu.VMEM((1024,), jnp.int32),
            pltpu.VMEM((1024,), jnp.int32),
            pltpu.VMEM((NBK,), jnp.int32),
            pltpu.VMEM((NBK,), jnp.int32),
            pltpu.VMEM((NBK,), jnp.int32),
            pltpu.VMEM((NBK,), jnp.int32),
            pltpu.VMEM((NBK,), jnp.int32),
            pltpu.VMEM((NBK * 16,), jnp.int32),
            pltpu.VMEM((NBK * 16,), jnp.int32),
            pltpu.VMEM((128,), jnp.int32),
            pltpu.VMEM((128,), jnp.int32),
            pltpu.VMEM((128,), jnp.int32),
            pltpu.VMEM((16,), jnp.int32),
            pltpu.SemaphoreType.DMA,
            pltpu.SemaphoreType.DMA,
        ],
    )


def _deg_body(sd_hbm, stw_hbm, deg_hbm, stb, sdb, cnt, degb):
    wid = _wid()
    iota = lax.iota(jnp.int32, 16)
    pltpu.sync_copy(stw_hbm.at[pl.ds(pl.multiple_of(wid * 16, 8), 16)], stb)
    sv = stb[...]
    for t in range(T):
        e0 = sv[2 * t]
        e1 = sv[2 * t + 1]

        def zero(i, _):
            cnt[pl.ds(pl.multiple_of(i * 16, 8), 16)] = jnp.zeros((16,), jnp.int32)
            return 0

        lax.fori_loop(0, RANGE + 1, zero, 0, unroll=False)
        nch = lax.div(e1 - e0 + 63, 64)

        def chunk(i, _):
            base = pl.multiple_of(e0 + i * 64, 8)
            pltpu.sync_copy(sd_hbm.at[pl.ds(base, 64)], sdb)
            for v in range(4):
                lanes = base + v * 16 + iota
                d = sdb[pl.ds(v * 16, 16)]
                ld = jnp.where(lanes < e1, d - wid * RANGE, RANGE)
                plsc.addupdate_scatter(cnt, [ld * 16 + iota], jnp.ones((16,), jnp.int32))
            return 0

        lax.fori_loop(0, nch, chunk, 0, unroll=False)
        for g in range(RANGE // 16):
            rr = g * 16 + iota
            acc = jnp.zeros((16,), jnp.int32)
            for L in range(16):
                acc = acc + plsc.load_gather(cnt, [rr * 16 + L])
            degb[pl.ds(g * 16, 16)] = (acc + 1).astype(jnp.float32)
        pltpu.sync_copy(degb, deg_hbm.at[pl.ds(pl.multiple_of(t * NPAD + wid * RANGE, 8), RANGE)])


def _make_deg():
    return pl.kernel(
        _deg_body,
        out_type=jax.ShapeDtypeStruct((T * NPAD,), jnp.float32),
        compiler_params=pltpu.CompilerParams(needs_layout_passes=False),
        mesh=_mesh(),
        scratch_types=[
            pltpu.VMEM((16,), jnp.int32),
            pltpu.VMEM((64,), jnp.int32),
            pltpu.VMEM(((RANGE + 1) * 16,), jnp.int32),
            pltpu.VMEM((RANGE,), jnp.float32),
        ],
    )


def _layer_body(gall_hbm, sg_hbm, sd_hbm, stw_hbm, m_hbm,
                stb, sgb, sdb, rows_v, acc, sem):
    wid = _wid()
    iota = lax.iota(jnp.int32, 16)
    pltpu.sync_copy(stw_hbm.at[pl.ds(pl.multiple_of(wid * 16, 8), 16)], stb)
    sv = stb[...]
    for t in range(T):
        e0 = sv[2 * t]
        e1 = sv[2 * t + 1]
        gbase = pl.multiple_of(t * NPAD + wid * RANGE, 8)
        pltpu.sync_copy(gall_hbm.at[pl.ds(gbase, RANGE)], acc.at[pl.ds(0, RANGE)])
        nch = lax.div(e1 - e0 + (CH - 1), CH)

        def chunk(i, _):
            base = pl.multiple_of(e0 + i * CH, 8)
            pltpu.sync_copy(sg_hbm.at[pl.ds(base, CH)], sgb)
            pltpu.sync_copy(sd_hbm.at[pl.ds(base, CH)], sdb)
            for v in range(CH // 16):
                lanes = base + v * 16 + iota
                sgv = sgb[pl.ds(v * 16, 16)]
                sgb[pl.ds(v * 16, 16)] = jnp.where(lanes < e1, sgv, 0)
            pltpu.async_copy(gall_hbm.at[sgb], rows_v, sem).wait()
            for v in range(CH // 16):
                lanes = base + v * 16 + iota
                ld = jnp.where(lanes < e1, sdb[pl.ds(v * 16, 16)] - wid * RANGE, RANGE)
                for j in range(16):
                    ldst = ld[j]
                    jj = v * 16 + j
                    for c in range(H // 16):
                        a = acc[ldst, pl.ds(c * 16, 16)]
                        g = rows_v[jj, pl.ds(c * 16, 16)]
                        acc[ldst, pl.ds(c * 16, 16)] = jnp.maximum(a, g)
            return 0

        lax.fori_loop(0, nch, chunk, 0, unroll=False)
        pltpu.sync_copy(acc.at[pl.ds(0, RANGE)], m_hbm.at[pl.ds(gbase, RANGE)])


def _make_layer():
    return pl.kernel(
        _layer_body,
        out_type=jax.ShapeDtypeStruct((T * NPAD, H), jnp.float32),
        mesh=_mesh(),
        scratch_types=[
            pltpu.VMEM((16,), jnp.int32),
            pltpu.VMEM((CH,), jnp.int32),
            pltpu.VMEM((CH,), jnp.int32),
            pltpu.VMEM((CH, H), jnp.float32),
            pltpu.VMEM((RANGE + 1, H), jnp.float32),
            pltpu.SemaphoreType.DMA,
        ],
    )


def _emb_tc(xp, maskp, emb_W, emb_b, mask_tok):
    def body(x_ref, m_ref, w_ref, b_ref, tok_ref, o_ref):
        h = jnp.dot(x_ref[...], w_ref[...], preferred_element_type=jnp.float32)
        h = h + b_ref[...]
        o_ref[...] = jnp.where(m_ref[...], tok_ref[...], h)

    return pl.pallas_call(
        body,
        grid=(NBLK,),
        in_specs=[
            pl.BlockSpec((BLK, F), lambda i: (i, 0)),
            pl.BlockSpec((BLK, 1), lambda i: (i, 0)),
            pl.BlockSpec((F, H), lambda i: (0, 0)),
            pl.BlockSpec((1, H), lambda i: (0, 0)),
            pl.BlockSpec((1, H), lambda i: (0, 0)),
        ],
        out_specs=pl.BlockSpec((BLK, H), lambda i: (i, 0)),
        out_shape=jax.ShapeDtypeStruct((NPAD, H), jnp.float32),
    )(xp, maskp, emb_W, emb_b[None, :], mask_tok[None, :])


def _dis_tc(deg):
    def body(d_ref, o_ref):
        o_ref[...] = lax.rsqrt(d_ref[...])

    return pl.pallas_call(
        body,
        out_shape=jax.ShapeDtypeStruct((T, NPAD), jnp.float32),
    )(deg)


def _gmat_tc(x, W4, dis):
    def body(x_ref, w_ref, d_ref, o_ref):
        g = jnp.dot(x_ref[...], w_ref[0], preferred_element_type=jnp.float32)
        o_ref[...] = g * d_ref[...]

    return pl.pallas_call(
        body,
        grid=(T, NBLK),
        in_specs=[
            pl.BlockSpec((BLK, H), lambda t, i: (i, 0)),
            pl.BlockSpec((1, H, H), lambda t, i: (t, 0, 0)),
            pl.BlockSpec((BLK, 1), lambda t, i: (t * NBLK + i, 0)),
        ],
        out_specs=pl.BlockSpec((BLK, H), lambda t, i: (t * NBLK + i, 0)),
        out_shape=jax.ShapeDtypeStruct((T * NPAD, H), jnp.float32),
    )(x, W4, dis.reshape(T * NPAD, 1))


def _s2_tc(x, M, dis, b4):
    def body(x_ref, m_ref, d_ref, b_ref, y_ref, st_ref):
        i = pl.program_id(0)
        y = x_ref[...]
        for t in range(T):
            y = y + d_ref[t][:, None] * m_ref[t] + b_ref[0, t][None, :]
        y_ref[...] = y
        rows = i * BLK + lax.broadcasted_iota(jnp.int32, (BLK, 1), 0)
        valid = (rows < N).astype(jnp.float32)
        yv = y * valid
        s1 = jnp.sum(yv, axis=0)
        s2 = jnp.sum(yv * yv, axis=0)
        st = jnp.concatenate(
            [s1[None], s2[None], jnp.zeros((6, H), jnp.float32)], axis=0)

        @pl.when(i == 0)
        def _():
            st_ref[...] = st

        @pl.when(i > 0)
        def _():
            st_ref[...] = st_ref[...] + st

    return pl.pallas_call(
        body,
        grid=(NBLK,),
        in_specs=[
            pl.BlockSpec((BLK, H), lambda i: (i, 0)),
            pl.BlockSpec((T, BLK, H), lambda i: (0, i, 0)),
            pl.BlockSpec((T, BLK), lambda i: (0, i)),
            pl.BlockSpec((1, T, H), lambda i: (0, 0, 0)),
        ],
        out_specs=[
            pl.BlockSpec((BLK, H), lambda i: (i, 0)),
            pl.BlockSpec((8, H), lambda i: (0, 0)),
        ],
        out_shape=[
            jax.ShapeDtypeStruct((NPAD, H), jnp.float32),
            jax.ShapeDtypeStruct((8, H), jnp.float32),
        ],
    )(x, M.reshape(T, NPAD, H), dis, b4[None])


def _s3_tc(y, st1, g1, b1, l1W, l1b, l2W, l2b):
    def body(y_ref, st_ref, g_ref, b_ref, w1_ref, b1_ref, w2_ref, b2_ref,
             u_ref, st2_ref):
        i = pl.program_id(0)
        m = st_ref[0] * (1.0 / N)
        var = st_ref[1] * (1.0 / N) - m * m
        inv = lax.rsqrt(var + EPS)
        xb = (y_ref[...] - m[None, :]) * inv[None, :] * g_ref[...] + b_ref[...]
        h1 = jnp.dot(xb, w1_ref[...], preferred_element_type=jnp.float32)
        h1 = jnp.maximum(h1 + b1_ref[...], 0.0)
        u = jnp.dot(h1, w2_ref[...], preferred_element_type=jnp.float32) + b2_ref[...]
        u_ref[...] = u
        rows = i * BLK + lax.broadcasted_iota(jnp.int32, (BLK, 1), 0)
        valid = (rows < N).astype(jnp.float32)
        uv = u * valid
        st = jnp.concatenate(
            [jnp.sum(uv, axis=0)[None], jnp.sum(uv * uv, axis=0)[None],
             jnp.zeros((6, H), jnp.float32)], axis=0)

        @pl.when(i == 0)
        def _():
            st2_ref[...] = st

        @pl.when(i > 0)
        def _():
            st2_ref[...] = st2_ref[...] + st

    return pl.pallas_call(
        body,
        grid=(NBLK,),
        in_specs=[
            pl.BlockSpec((BLK, H), lambda i: (i, 0)),
            pl.BlockSpec((8, H), lambda i: (0, 0)),
            pl.BlockSpec((1, H), lambda i: (0, 0)),
            pl.BlockSpec((1, H), lambda i: (0, 0)),
            pl.BlockSpec((H, DI), lambda i: (0, 0)),
            pl.BlockSpec((1, DI), lambda i: (0, 0)),
            pl.BlockSpec((DI, H), lambda i: (0, 0)),
            pl.BlockSpec((1, H), lambda i: (0, 0)),
        ],
        out_specs=[
            pl.BlockSpec((BLK, H), lambda i: (i, 0)),
            pl.BlockSpec((8, H), lambda i: (0, 0)),
        ],
        out_shape=[
            jax.ShapeDtypeStruct((NPAD, H), jnp.float32),
            jax.ShapeDtypeStruct((8, H), jnp.float32),
        ],
    )(y, st1, g1[None, :], b1[None, :], l1W, l1b[None, :], l2W, l2b[None, :])


def _s4_tc(u, st2, g2, b2, res, W4, dis):
    has_res = res is not None

    def body(*refs):
        if has_res:
            (u_ref, st_ref, g_ref, b_ref, r_ref, w_ref, d_ref, xn_ref, go_ref) = refs
        else:
            (u_ref, st_ref, g_ref, b_ref, w_ref, d_ref, xn_ref, go_ref) = refs
        t = pl.program_id(0)
        m = st_ref[0] * (1.0 / N)
        var = st_ref[1] * (1.0 / N) - m * m
        inv = lax.rsqrt(var + EPS)
        xn = jnp.maximum((u_ref[...] - m[None, :]) * inv[None, :] * g_ref[...] + b_ref[...], 0.0)
        if has_res:
            xn = xn + r_ref[...]
        xn_ref[...] = xn
        g = jnp.dot(xn, w_ref[0], preferred_element_type=jnp.float32)
        go_ref[...] = g * d_ref[...]

    in_specs = [
        pl.BlockSpec((BLK, H), lambda t, i: (i, 0)),
        pl.BlockSpec((8, H), lambda t, i: (0, 0)),
        pl.BlockSpec((1, H), lambda t, i: (0, 0)),
        pl.BlockSpec((1, H), lambda t, i: (0, 0)),
    ]
    args = [u, st2, g2[None, :], b2[None, :]]
    if has_res:
        in_specs.append(pl.BlockSpec((BLK, H), lambda t, i: (i, 0)))
        args.append(res)
    in_specs += [
        pl.BlockSpec((1, H, H), lambda t, i: (t, 0, 0)),
        pl.BlockSpec((BLK, 1), lambda t, i: (t * NBLK + i, 0)),
    ]
    args += [W4, dis.reshape(T * NPAD, 1)]
    return pl.pallas_call(
        body,
        grid=(T, NBLK),
        in_specs=in_specs,
        out_specs=[
            pl.BlockSpec((BLK, H), lambda t, i: (i, 0)),
            pl.BlockSpec((BLK, H), lambda t, i: (t * NBLK + i, 0)),
        ],
        out_shape=[
            jax.ShapeDtypeStruct((NPAD, H), jnp.float32),
            jax.ShapeDtypeStruct((T * NPAD, H), jnp.float32),
        ],
    )(*args)


def _s4_last_tc(u, st2, g2, b2, res):
    def body(u_ref, st_ref, g_ref, b_ref, r_ref, o_ref):
        m = st_ref[0] * (1.0 / N)
        var = st_ref[1] * (1.0 / N) - m * m
        inv = lax.rsqrt(var + EPS)
        xn = jnp.maximum((u_ref[...] - m[None, :]) * inv[None, :] * g_ref[...] + b_ref[...], 0.0)
        o_ref[...] = xn + r_ref[...]

    return pl.pallas_call(
        body,
        grid=(NBLK,),
        in_specs=[
            pl.BlockSpec((BLK, H), lambda i: (i, 0)),
            pl.BlockSpec((8, H), lambda i: (0, 0)),
            pl.BlockSpec((1, H), lambda i: (0, 0)),
            pl.BlockSpec((1, H), lambda i: (0, 0)),
            pl.BlockSpec((BLK, H), lambda i: (i, 0)),
        ],
        out_specs=pl.BlockSpec((BLK, H), lambda i: (i, 0)),
        out_shape=jax.ShapeDtypeStruct((NPAD, H), jnp.float32),
    )(u, st2, g2[None, :], b2[None, :], res)


def kernel(x, mask, edge_index, edge_type, reliable_masking, emb_W, emb_b, mask_tok,
           enc_gcn_W, enc_gcn_b, enc_bn1_g, enc_bn1_b, enc_bn2_g, enc_bn2_b,
           enc_l1_W, enc_l1_b, enc_l2_W, enc_l2_b,
           dec_gcn_W, dec_gcn_b, dec_bn1_g, dec_bn1_b, dec_bn2_g, dec_bn2_b,
           dec_l1_W, dec_l1_b, dec_l2_W, dec_l2_b):
    E = edge_index.shape[1]
    pad_e = EP - E
    srcp = jnp.concatenate([edge_index[0], jnp.full((pad_e,), NPAD - 1, jnp.int32)])
    dstp = jnp.concatenate([edge_index[1], jnp.full((pad_e,), NPAD - 2, jnp.int32)])
    etp = jnp.concatenate([edge_type, jnp.zeros((pad_e,), jnp.int32)])

    tot = _make_sortA()(dstp, etp)
    sg, sd, stw = _make_sortB()(dstp, etp, srcp, tot)
    deg = _make_deg()(sd, stw)
    dis = _dis_tc(deg.reshape(T, NPAD))

    xp = jnp.pad(x[:, 0, :], ((0, NPAD - N), (0, 0)))
    maskp = jnp.pad(mask, ((0, NPAD - N), (0, 0)))
    h0 = _emb_tc(xp, maskp, emb_W, emb_b, mask_tok)

    noise = jax.random.normal(jax.random.key(42), (N, H), dtype=jnp.float32)
    nmask = jnp.zeros((H,), jnp.float32).at[: H // 2].set(1.0)
    noisep = jnp.pad(nmask[None, :] * noise, ((0, NPAD - N), (0, 0)))

    layers = []
    for i in range(2):
        layers.append((enc_gcn_W[i], enc_gcn_b[i], enc_bn1_g[i], enc_bn1_b[i],
                       enc_bn2_g[i], enc_bn2_b[i], enc_l1_W[i], enc_l1_b[i],
                       enc_l2_W[i], enc_l2_b[i]))
    for _ in range(NUM_ITER):
        for i in range(2):
            layers.append((dec_gcn_W[i], dec_gcn_b[i], dec_bn1_g[i], dec_bn1_b[i],
                           dec_bn2_g[i], dec_bn2_b[i], dec_l1_W[i], dec_l1_b[i],
                           dec_l2_W[i], dec_l2_b[i]))

    sc_layer = _make_layer()
    x_cur = h0
    gall = _gmat_tc(x_cur, layers[0][0], dis)
    stack_in = None
    for l in range(8):
        (W4, b4, g1, b1, g2, b2, l1W, l1b, l2W, l2b) = layers[l]
        if l in (2, 4, 6):
            stack_in = x_cur
        M = sc_layer(gall, sg, sd, stw)
        y, st1 = _s2_tc(x_cur, M, dis, b4)
        u, st2 = _s3_tc(y, st1, g1, b1, l1W, l1b, l2W, l2b)
        if l == 7:
            x_cur = _s4_last_tc(u, st2, g2, b2, stack_in)
        else:
            res = None
            if l == 1:
                res = noisep
            elif l in (3, 5):
                res = stack_in
            x_cur, gall = _s4_tc(u, st2, g2, b2, res, layers[l + 1][0], dis)
    return x_cur[:N]
